# baseline (speedup 1.0000x reference)
import functools

import jax
import jax.numpy as jnp
import numpy as np
from jax.experimental import pallas as pl
from jax.experimental.pallas import tpu as pltpu
from jax.sharding import Mesh, PartitionSpec as P


def _mil_step(x_ref, w1_ref, b1_ref, w2_ref, b2_ref, inst_ref, bag_ref,
              *, n_inst, bt):
    h = jnp.dot(x_ref[...], w1_ref[...], preferred_element_type=jnp.float32)
    h = jnp.maximum(h + b1_ref[...], 0.0)
    inst = jnp.dot(h, w2_ref[...], preferred_element_type=jnp.float32)
    inst = (inst + b2_ref[...])[:, :inst_ref.shape[-1]]
    inst_ref[...] = inst
    c = inst_ref.shape[-1]
    bag_ref[...] = jnp.sum(inst.reshape(bt, n_inst, c), axis=1) * (
        jnp.float32(1.0 / n_inst))


def _mil_block(x2d, w1, b1r, w2p, b2r, *, n_inst, n_classes, bt):
    rows_total, D = x2d.shape
    H = w1.shape[1]
    cp = w2p.shape[1]
    rows = bt * n_inst
    const = lambda i: (0, 0)
    blk = lambda i: (i, 0)
    return pl.pallas_call(
        functools.partial(_mil_step, n_inst=n_inst, bt=bt),
        grid=(rows_total // rows,),
        in_specs=[
            pl.BlockSpec((rows, D), blk),
            pl.BlockSpec((D, H), const),
            pl.BlockSpec((1, H), const),
            pl.BlockSpec((H, cp), const),
            pl.BlockSpec((1, cp), const),
        ],
        out_specs=[
            pl.BlockSpec((rows, n_classes), blk),
            pl.BlockSpec((bt, n_classes), blk),
        ],
        out_shape=(
            jax.ShapeDtypeStruct((rows_total, n_classes), jnp.float32),
            jax.ShapeDtypeStruct((rows_total // n_inst, n_classes),
                                 jnp.float32),
        ),
        compiler_params=pltpu.CompilerParams(
            dimension_semantics=("parallel",)),
    )(x2d, w1, b1r, w2p, b2r)


def kernel(bags, w1, b1, w2, b2):
    B, N, D = bags.shape
    H = w1.shape[1]
    C = w2.shape[1]

    x2d = bags.reshape(B * N, D).astype(jnp.float32)
    b1r = b1.reshape(1, H).astype(jnp.float32)
    cp = max(256, C)
    w2p = jnp.zeros((H, cp), jnp.float32).at[:, :C].set(w2)
    b2r = jnp.zeros((1, cp), jnp.float32).at[:, :C].set(b2.reshape(1, C))

    block = functools.partial(_mil_block, n_inst=N, n_classes=C)
    devs = [d for d in jax.devices() if d.platform == "tpu"][:2]
    if len(devs) == 2 and B % 128 == 0:
        mesh = Mesh(np.asarray(devs), ("x",))
        fn = jax.shard_map(
            functools.partial(block, bt=32),
            mesh=mesh,
            in_specs=(P("x", None), P(None, None), P(None, None),
                      P(None, None), P(None, None)),
            out_specs=(P("x", None), P("x", None)),
            check_vma=False,
        )
        inst2d, bag_preds = fn(x2d, w1, b1r, w2p, b2r)
    else:
        inst2d, bag_preds = block(x2d, w1, b1r, w2p, b2r, bt=64)
    return bag_preds, inst2d.reshape(B, N, C)


# speedup vs baseline: 12.6374x; 12.6374x over previous
import functools

import jax
import jax.numpy as jnp
from jax.experimental import pallas as pl
from jax.experimental.pallas import tpu as pltpu

_BT = 64


def _mil_step(x_ref, w1_ref, b1_ref, w2_ref, b2_ref, inst_ref, bag_ref,
              *, n_inst):
    rows = _BT * n_inst
    h = jnp.dot(x_ref[...], w1_ref[...], preferred_element_type=jnp.float32)
    h = jnp.maximum(h + b1_ref[...], 0.0)
    inst = jnp.dot(h, w2_ref[...], preferred_element_type=jnp.float32)
    inst = (inst + b2_ref[...])[:, :inst_ref.shape[-1]]
    inst_ref[...] = inst
    c = inst_ref.shape[-1]
    bag_ref[...] = jnp.sum(inst.reshape(_BT, n_inst, c), axis=1) * (
        jnp.float32(1.0 / n_inst))


def kernel(bags, w1, b1, w2, b2):
    B, N, D = bags.shape
    H = w1.shape[1]
    C = w2.shape[1]
    bt = _BT
    assert B % bt == 0
    rows = bt * N

    x2d = bags.reshape(B * N, D).astype(jnp.float32)
    b1r = b1.reshape(1, H).astype(jnp.float32)
    cp = max(256, C)
    w2p = jnp.zeros((H, cp), jnp.float32).at[:, :C].set(w2)
    b2r = jnp.zeros((1, cp), jnp.float32).at[:, :C].set(b2.reshape(1, C))

    const = lambda i: (0, 0)
    blk = lambda i: (i, 0)
    inst2d, bag_preds = pl.pallas_call(
        functools.partial(_mil_step, n_inst=N),
        grid=(B // bt,),
        in_specs=[
            pl.BlockSpec((rows, D), blk),
            pl.BlockSpec((D, H), const),
            pl.BlockSpec((1, H), const),
            pl.BlockSpec((H, cp), const),
            pl.BlockSpec((1, cp), const),
        ],
        out_specs=[
            pl.BlockSpec((rows, C), blk),
            pl.BlockSpec((bt, C), blk),
        ],
        out_shape=(
            jax.ShapeDtypeStruct((B * N, C), jnp.float32),
            jax.ShapeDtypeStruct((B, C), jnp.float32),
        ),
        compiler_params=pltpu.CompilerParams(
            dimension_semantics=("parallel",)),
    )(x2d, w1, b1r, w2p, b2r)
    return bag_preds, inst2d.reshape(B, N, C)


# speedup vs baseline: 15.4823x; 1.2251x over previous
import functools

import jax
import jax.numpy as jnp
from jax.experimental import pallas as pl
from jax.experimental.pallas import tpu as pltpu

_BT = 64


def _mil_step(x_ref, w1_ref, b1_ref, w2_ref, b2_ref, inst_ref, bag_ref,
              *, n_inst):
    rows = _BT * n_inst
    h = jnp.dot(x_ref[...], w1_ref[...], preferred_element_type=jnp.float32)
    h = jnp.maximum(h + b1_ref[...], 0.0)
    inst = jnp.dot(h, w2_ref[...], preferred_element_type=jnp.float32)
    inst = inst + b2_ref[...]
    inst_ref[...] = inst
    c = inst_ref.shape[-1]
    bag_ref[...] = jnp.sum(inst.reshape(_BT, n_inst, c), axis=1) * (
        jnp.float32(1.0 / n_inst))


def kernel(bags, w1, b1, w2, b2):
    B, N, D = bags.shape
    H = w1.shape[1]
    C = w2.shape[1]
    bt = _BT
    assert B % bt == 0
    rows = bt * N

    x2d = bags.reshape(B * N, D).astype(jnp.float32)
    b1r = b1.reshape(1, H).astype(jnp.float32)
    b2r = b2.reshape(1, C).astype(jnp.float32)

    const = lambda i: (0, 0)
    blk = lambda i: (i, 0)
    inst2d, bag_preds = pl.pallas_call(
        functools.partial(_mil_step, n_inst=N),
        grid=(B // bt,),
        in_specs=[
            pl.BlockSpec((rows, D), blk),
            pl.BlockSpec((D, H), const),
            pl.BlockSpec((1, H), const),
            pl.BlockSpec((H, C), const),
            pl.BlockSpec((1, C), const),
        ],
        out_specs=[
            pl.BlockSpec((rows, C), blk),
            pl.BlockSpec((bt, C), blk),
        ],
        out_shape=(
            jax.ShapeDtypeStruct((B * N, C), jnp.float32),
            jax.ShapeDtypeStruct((B, C), jnp.float32),
        ),
        compiler_params=pltpu.CompilerParams(
            dimension_semantics=("parallel",)),
    )(x2d, w1, b1r, w2, b2r)
    return bag_preds, inst2d.reshape(B, N, C)
